# baseline (speedup 1.0000x reference)
import functools

import jax
import jax.numpy as jnp
from jax import lax
from jax.experimental import pallas as pl
from jax.experimental.pallas import tpu as pltpu
from jax.experimental.pallas import tpu_sc as plsc

_N_NODES = 100000
_N_EDGES = 3200000
_MAX_NODES = 10000
_OUT = 64

_NC = 2
_NS = 16
_NW = _NC * _NS
_L = 16

_HROWS = 800
_UNIT = 1024
_UNITS = _N_EDGES // _UNIT
_BASE_U = _UNITS // _NW
_EXTRA = _UNITS - _BASE_U * _NW
_CHUNK_U = 7
_CHUNK_E = _CHUNK_U * _UNIT
_TAIL_U = _BASE_U % _CHUNK_U
_TAIL_E = _TAIL_U * _UNIT
_UNROLL = 8


def _sc_hist_body(edge_hbm, zeros_hbm, out_hbm, idx_v, hist_v):
    c = lax.axis_index("c")
    s = lax.axis_index("s")
    w = c * _NS + s
    pltpu.sync_copy(zeros_hbm, hist_v)
    start_u = _BASE_U * w + jnp.minimum(w, _EXTRA)
    n_chunks = jnp.where(w < _EXTRA, 14, 13)
    ones = jnp.ones((_L,), jnp.float32)

    def _scat_vregs(n_vregs):
        def scat(g, _):
            for u in range(_UNROLL):
                idx = idx_v[pl.ds((g * _UNROLL + u) * _L, _L)]
                plsc.addupdate_scatter(
                    hist_v,
                    [lax.shift_right_logical(idx, 7),
                     lax.bitwise_and(idx, 127)],
                    ones)
            return 0

        lax.fori_loop(0, n_vregs // _UNROLL, scat, 0)

    def chunk(ci, _):
        off = (start_u + ci * _CHUNK_U) * _UNIT
        pltpu.sync_copy(edge_hbm.at[pl.ds(off, _CHUNK_E)], idx_v)
        _scat_vregs(_CHUNK_E // _L)
        return 0

    lax.fori_loop(0, n_chunks, chunk, 0)

    @pl.when(w >= _EXTRA)
    def _tail():
        off = (start_u + 13 * _CHUNK_U) * _UNIT
        pltpu.sync_copy(edge_hbm.at[pl.ds(off, _TAIL_E)],
                        idx_v.at[pl.ds(0, _TAIL_E)])
        _scat_vregs(_TAIL_E // _L)

    pltpu.sync_copy(hist_v, out_hbm.at[w])


@functools.cache
def _sc_histogram():
    mesh = plsc.VectorSubcoreMesh(
        core_axis_name="c", subcore_axis_name="s", num_cores=_NC,
        num_subcores=_NS,
    )
    return pl.kernel(
        _sc_hist_body,
        out_type=jax.ShapeDtypeStruct((_NW, _HROWS, 128), jnp.float32),
        mesh=mesh,
        compiler_params=pltpu.CompilerParams(needs_layout_passes=False),
        scratch_types=[
            pltpu.VMEM((_CHUNK_E,), jnp.int32),
            pltpu.VMEM((_HROWS, 128), jnp.float32),
        ],
    )


_BLK = 2048
_TGRID = -(-_N_NODES // _BLK)


def _tc_body(sc, emb, W1, b1, Wd1, bd1, Wd2, bd2, Wcn, Wcd, bc, gamma, beta,
             out, A2, eye):
    f32 = jnp.float32
    i = pl.program_id(0)

    @pl.when(i == 0)
    def _init():
        ne = jnp.maximum(emb[...], 0.0)
        ne = jnp.dot(ne, W1[...], preferred_element_type=f32) + b1[...]
        A = jnp.dot(ne, Wcn[...], preferred_element_type=f32) + bc[...]
        A2[pl.ds(0, _MAX_NODES), :] = A
        A2[pl.ds(_MAX_NODES, _MAX_NODES), :] = A
        r = lax.broadcasted_iota(jnp.int32, (128, 128), 0)
        c = lax.broadcasted_iota(jnp.int32, (128, 128), 1)
        eye[...] = (r == c).astype(f32)

    dsum = jnp.sum(sc[...], axis=0)
    t1 = lax.dot_general(eye[...], dsum, (((0,), (1,)), ((), ())),
                         preferred_element_type=f32)
    D16 = jnp.concatenate(
        [jnp.broadcast_to(t1[:, r:r + 1], (128, _L)) for r in range(16)],
        axis=0)
    T = jnp.maximum(D16 * Wd1[...] + bd1[...], 0.0)
    de = jnp.dot(T, Wd2[...], preferred_element_type=f32) + bd2[...]
    G = jnp.dot(de, Wcd[...], preferred_element_type=f32)
    start = lax.rem(i * _BLK, _MAX_NODES)
    a = A2[pl.ds(start, _BLK), :]
    h = jnp.maximum(a + G, 0.0)
    m = jnp.mean(h, axis=1, keepdims=True)
    hc = h - m
    v = jnp.mean(hc * hc, axis=1, keepdims=True)
    out[...] = hc * lax.rsqrt(v + 1e-5) * gamma[...] + beta[...]


def _full(shape):
    return pl.BlockSpec(shape, lambda i: (0,) * len(shape))


_tc_dense = pl.pallas_call(
    _tc_body,
    grid=(_TGRID,),
    in_specs=[
        pl.BlockSpec((_NW, 16, 128), lambda i: (0, i, 0)),
        _full((_MAX_NODES, 32)),
        _full((32, _OUT)),
        _full((1, _OUT)),
        _full((1, 16)),
        _full((1, 16)),
        _full((16, 32)),
        _full((1, 32)),
        _full((_OUT, _OUT)),
        _full((32, _OUT)),
        _full((1, _OUT)),
        _full((1, _OUT)),
        _full((1, _OUT)),
    ],
    out_specs=pl.BlockSpec((_BLK, _OUT), lambda i: (i, 0)),
    out_shape=jax.ShapeDtypeStruct((_N_NODES, _OUT), jnp.float32),
    scratch_shapes=[
        pltpu.VMEM((2 * _MAX_NODES, _OUT), jnp.float32),
        pltpu.VMEM((128, 128), jnp.float32),
    ],
)


def kernel(edge_index, emb, W1, b1, Wd1, bd1, Wd2, bd2, Wc, bc, gamma, beta):
    zeros = jnp.zeros((_HROWS, 128), jnp.float32)
    hists = _sc_histogram()(edge_index[0], zeros)
    return _tc_dense(
        hists, emb, W1, b1.reshape(1, -1), Wd1, bd1.reshape(1, -1), Wd2,
        bd2.reshape(1, -1), Wc[:_OUT], Wc[_OUT:], bc.reshape(1, -1),
        gamma.reshape(1, -1), beta.reshape(1, -1),
    )


# speedup vs baseline: 1.3111x; 1.3111x over previous
import functools

import jax
import jax.numpy as jnp
from jax import lax
from jax.experimental import pallas as pl
from jax.experimental.pallas import tpu as pltpu
from jax.experimental.pallas import tpu_sc as plsc

_N_NODES = 100000
_N_EDGES = 3200000
_MAX_NODES = 10000
_OUT = 64

_NC = 2
_NS = 16
_NW = _NC * _NS
_L = 16

_HROWS = 800
_UNIT = 1024
_UNITS = _N_EDGES // _UNIT
_BASE_U = _UNITS // _NW
_EXTRA = _UNITS - _BASE_U * _NW
_CHUNK_U = 7
_CHUNK_E = _CHUNK_U * _UNIT
_TAIL_U = _BASE_U % _CHUNK_U
_TAIL_E = _TAIL_U * _UNIT
_UNROLL = 16


def _sc_hist_body(edge_hbm, out_hbm, idx_v, hist_v):
    c = lax.axis_index("c")
    s = lax.axis_index("s")
    w = c * _NS + s
    zero = jnp.zeros((_L,), jnp.float32)

    def _zrow(r, _):
        for l in range(128 // _L):
            hist_v[r, pl.ds(l * _L, _L)] = zero
        return 0

    lax.fori_loop(0, _HROWS, _zrow, 0)
    start_u = _BASE_U * w + jnp.minimum(w, _EXTRA)
    n_chunks = jnp.where(w < _EXTRA, 14, 13)
    ones = jnp.ones((_L,), jnp.float32)

    def _scat_vregs(n_vregs):
        def scat(g, _):
            vs = [idx_v[pl.ds((g * _UNROLL + u) * _L, _L)]
                  for u in range(_UNROLL)]
            for idx in vs:
                plsc.addupdate_scatter(
                    hist_v,
                    [lax.shift_right_logical(idx, 7),
                     lax.bitwise_and(idx, 127)],
                    ones)
            return 0

        lax.fori_loop(0, n_vregs // _UNROLL, scat, 0)

    def chunk(ci, _):
        off = (start_u + ci * _CHUNK_U) * _UNIT
        pltpu.sync_copy(edge_hbm.at[pl.ds(off, _CHUNK_E)], idx_v)
        _scat_vregs(_CHUNK_E // _L)
        return 0

    lax.fori_loop(0, n_chunks, chunk, 0)

    @pl.when(w >= _EXTRA)
    def _tail():
        off = (start_u + 13 * _CHUNK_U) * _UNIT
        pltpu.sync_copy(edge_hbm.at[pl.ds(off, _TAIL_E)],
                        idx_v.at[pl.ds(0, _TAIL_E)])
        _scat_vregs(_TAIL_E // _L)

    pltpu.sync_copy(hist_v, out_hbm.at[w])


@functools.cache
def _sc_histogram():
    mesh = plsc.VectorSubcoreMesh(
        core_axis_name="c", subcore_axis_name="s", num_cores=_NC,
        num_subcores=_NS,
    )
    return pl.kernel(
        _sc_hist_body,
        out_type=jax.ShapeDtypeStruct((_NW, _HROWS, 128), jnp.float32),
        mesh=mesh,
        compiler_params=pltpu.CompilerParams(needs_layout_passes=False),
        scratch_types=[
            pltpu.VMEM((_CHUNK_E,), jnp.int32),
            pltpu.VMEM((_HROWS, 128), jnp.float32),
        ],
    )


_BLK = 4096
_TGRID = -(-_N_NODES // _BLK)


def _tc_body(sc, emb, W1, b1, Wd1, bd1, Wd2, bd2, Wcn, Wcd, bc, gamma, beta,
             out, A2, eye):
    f32 = jnp.float32
    i = pl.program_id(0)

    @pl.when(i == 0)
    def _init():
        ne = jnp.maximum(emb[...], 0.0)
        ne = jnp.dot(ne, W1[...], preferred_element_type=f32) + b1[...]
        A = jnp.dot(ne, Wcn[...], preferred_element_type=f32) + bc[...]
        A2[pl.ds(0, _MAX_NODES), :] = A
        A2[pl.ds(_MAX_NODES, _MAX_NODES), :] = A
        r = lax.broadcasted_iota(jnp.int32, (128, 128), 0)
        c = lax.broadcasted_iota(jnp.int32, (128, 128), 1)
        eye[...] = (r == c).astype(f32)

    dsum = jnp.sum(sc[...], axis=0)
    t1 = lax.dot_general(eye[...], dsum, (((0,), (1,)), ((), ())),
                         preferred_element_type=f32)
    D16 = jnp.concatenate(
        [jnp.broadcast_to(t1[:, r:r + 1], (128, _L))
         for r in range(_BLK // 128)],
        axis=0)
    T = jnp.maximum(D16 * Wd1[...] + bd1[...], 0.0)
    de = jnp.dot(T, Wd2[...], preferred_element_type=f32) + bd2[...]
    G = jnp.dot(de, Wcd[...], preferred_element_type=f32)
    start = lax.rem(i * _BLK, _MAX_NODES)
    a = A2[pl.ds(start, _BLK), :]
    h = jnp.maximum(a + G, 0.0)
    m = jnp.mean(h, axis=1, keepdims=True)
    hc = h - m
    v = jnp.mean(hc * hc, axis=1, keepdims=True)
    out[...] = hc * lax.rsqrt(v + 1e-5) * gamma[...] + beta[...]


def _full(shape):
    return pl.BlockSpec(shape, lambda i: (0,) * len(shape))


_tc_dense = pl.pallas_call(
    _tc_body,
    grid=(_TGRID,),
    in_specs=[
        pl.BlockSpec((_NW, _BLK // 128, 128), lambda i: (0, i, 0)),
        _full((_MAX_NODES, 32)),
        _full((32, _OUT)),
        _full((1, _OUT)),
        _full((1, 16)),
        _full((1, 16)),
        _full((16, 32)),
        _full((1, 32)),
        _full((_OUT, _OUT)),
        _full((32, _OUT)),
        _full((1, _OUT)),
        _full((1, _OUT)),
        _full((1, _OUT)),
    ],
    out_specs=pl.BlockSpec((_BLK, _OUT), lambda i: (i, 0)),
    out_shape=jax.ShapeDtypeStruct((_N_NODES, _OUT), jnp.float32),
    scratch_shapes=[
        pltpu.VMEM((2 * _MAX_NODES, _OUT), jnp.float32),
        pltpu.VMEM((128, 128), jnp.float32),
    ],
)


def kernel(edge_index, emb, W1, b1, Wd1, bd1, Wd2, bd2, Wc, bc, gamma, beta):
    hists = _sc_histogram()(edge_index[0])
    return _tc_dense(
        hists, emb, W1, b1.reshape(1, -1), Wd1, bd1.reshape(1, -1), Wd2,
        bd2.reshape(1, -1), Wc[:_OUT], Wc[_OUT:], bc.reshape(1, -1),
        gamma.reshape(1, -1), beta.reshape(1, -1),
    )
